# baseline (device time: 110862 ns/iter reference)
import jax
import jax.numpy as jnp
from jax import lax
from jax.experimental import pallas as pl
from jax.experimental.pallas import tpu as pltpu

N_DEV = 4


def kernel(x, Wg, Wu, Wd):
    m, d_model = x.shape
    hid = Wg.shape[1]
    wp = hid // 4
    chunk = m // N_DEV
    dp = chunk
    half = d_model // 2
    n_rs = N_DEV - 1
    n_ag = N_DEV - 1

    def body(x_hbm, wg_hbm, wu_hbm, wd_hbm, out_ref,
             wg_bf, wu_bf, wd_bf, stage_w, stage_d, stage_x,
             wsem, dsem, xsem,
             recv_buf, send_sems, recv_sems, send_sems_l, recv_sems_l,
             ag_ssem_r, ag_rsem_r, ag_ssem_l, ag_rsem_l):
        my = lax.axis_index("i")
        left = lax.rem(my + N_DEV - 1, N_DEV)
        right = lax.rem(my + 1, N_DEV)

        barrier_sem = pltpu.get_barrier_semaphore()
        for nbr in (left, right):
            pl.semaphore_signal(
                barrier_sem, inc=1,
                device_id=(nbr,), device_id_type=pl.DeviceIdType.MESH,
            )
        pl.semaphore_wait(barrier_sem, 2)

        def w_dma(idx):
            ref = wg_hbm if idx < 4 else wu_hbm
            j = idx % 4
            return pltpu.make_async_copy(
                ref.at[:, j * wp:(j + 1) * wp], stage_w.at[idx % 2],
                wsem.at[idx % 2])

        def w_convert(idx):
            dst = wg_bf if idx < 4 else wu_bf
            j = idx % 4
            dst[:, j * wp:(j + 1) * wp] = stage_w[idx % 2].astype(jnp.bfloat16)

        def d_dma(idx):
            return pltpu.make_async_copy(
                wd_hbm.at[idx * dp:(idx + 1) * dp, :], stage_d.at[idx % 2],
                dsem.at[idx % 2])

        def d_convert(idx):
            wd_bf[idx * dp:(idx + 1) * dp, :] = (
                stage_d[idx % 2].astype(jnp.bfloat16))

        def x_dma(c, k):
            return pltpu.make_async_copy(
                x_hbm.at[pl.ds(c * chunk, chunk), :], stage_x.at[k % 2],
                xsem.at[k % 2])

        chunk_order = [
            my,
            lax.rem(my - 1 + N_DEV, N_DEV),
            lax.rem(my + 1, N_DEV),
            lax.rem(my + 2, N_DEV),
        ]

        x_dma(chunk_order[0], 0).start()
        x_dma(chunk_order[1], 1).start()
        w_dma(0).start()
        w_dma(1).start()
        d_dma(0).start()
        d_dma(1).start()
        for idx in range(4):
            w_dma(idx).wait()
            w_convert(idx)
            if idx + 2 < 8:
                w_dma(idx + 2).start()

        def compute_chunk(k):
            c = chunk_order[k]
            x_dma(c, k).wait()
            xc = stage_x[k % 2].astype(jnp.bfloat16)
            if k + 2 < N_DEV:
                x_dma(chunk_order[k + 2], k + 2).start()
            gate = jnp.dot(xc, wg_bf[:, :], preferred_element_type=jnp.float32)
            if k == 0:
                for idx in range(4, 8):
                    w_dma(idx).wait()
                    w_convert(idx)
                    if idx + 2 < 8:
                        w_dma(idx + 2).start()
            up = jnp.dot(xc, wu_bf[:, :], preferred_element_type=jnp.float32)
            h = (gate * (up * jax.nn.sigmoid(up))).astype(jnp.bfloat16)
            if k == 0:
                for idx in range(8):
                    d_dma(idx).wait()
                    d_convert(idx)
                    if idx + 2 < 8:
                        d_dma(idx + 2).start()
            out_ref[pl.ds(c * chunk, chunk), :] = jnp.dot(
                h, wd_bf[:, :], preferred_element_type=jnp.float32
            ).astype(jnp.bfloat16)

        def rs_start(s, rightward):
            if rightward:
                c = lax.rem(my - s + N_DEV, N_DEV)
                src = out_ref.at[pl.ds(c * chunk, chunk), 0:half]
                dst = recv_buf.at[s]
                dev, ssem, rsem = right, send_sems, recv_sems
            else:
                c = lax.rem(my + s, N_DEV)
                src = out_ref.at[pl.ds(c * chunk, chunk), half:]
                dst = recv_buf.at[n_rs + s]
                dev, ssem, rsem = left, send_sems_l, recv_sems_l
            rdma = pltpu.make_async_remote_copy(
                src_ref=src, dst_ref=dst,
                send_sem=ssem.at[s], recv_sem=rsem.at[s],
                device_id=(dev,), device_id_type=pl.DeviceIdType.MESH,
            )
            rdma.start()
            return rdma

        def rs_finish(rdma, s, rightward):
            rdma.wait()
            if rightward:
                c = lax.rem(my - s - 1 + N_DEV, N_DEV)
                dst = (pl.ds(c * chunk, chunk), slice(0, half))
                got = recv_buf[s, :, :]
            else:
                c = lax.rem(my + s + 1, N_DEV)
                dst = (pl.ds(c * chunk, chunk), slice(half, d_model))
                got = recv_buf[n_rs + s, :, :]
            out_ref[dst] = out_ref[dst] + got

        compute_chunk(0)
        rs_r0 = rs_start(0, True)
        rs_l0 = rs_start(0, False)
        compute_chunk(1)
        rs_finish(rs_r0, 0, True)
        rs_r1 = rs_start(1, True)
        compute_chunk(2)
        rs_finish(rs_l0, 0, False)
        rs_l1 = rs_start(1, False)
        compute_chunk(3)
        rs_finish(rs_r1, 1, True)
        rs_r2 = rs_start(2, True)
        rs_finish(rs_l1, 1, False)
        rs_l2 = rs_start(2, False)
        rs_finish(rs_r2, 2, True)
        rs_finish(rs_l2, 2, False)

        for s in range(n_ag):
            c_r = lax.rem(my + 1 - s + N_DEV, N_DEV)
            c_l = lax.rem(my - 1 + s + N_DEV, N_DEV)
            rows_r = pl.ds(c_r * chunk, chunk)
            rows_l = pl.ds(c_l * chunk, chunk)
            rdma_r = pltpu.make_async_remote_copy(
                src_ref=out_ref.at[rows_r, 0:half],
                dst_ref=out_ref.at[rows_r, 0:half],
                send_sem=ag_ssem_r.at[s],
                recv_sem=ag_rsem_r.at[s],
                device_id=(right,),
                device_id_type=pl.DeviceIdType.MESH,
            )
            rdma_l = pltpu.make_async_remote_copy(
                src_ref=out_ref.at[rows_l, half:],
                dst_ref=out_ref.at[rows_l, half:],
                send_sem=ag_ssem_l.at[s],
                recv_sem=ag_rsem_l.at[s],
                device_id=(left,),
                device_id_type=pl.DeviceIdType.MESH,
            )
            rdma_r.start()
            rdma_l.start()
            rdma_r.wait()
            rdma_l.wait()

    return pl.pallas_call(
        body,
        out_shape=jax.ShapeDtypeStruct((m, d_model), jnp.bfloat16),
        in_specs=[pl.BlockSpec(memory_space=pl.ANY)] * 4,
        out_specs=pl.BlockSpec(memory_space=pltpu.VMEM),
        scratch_shapes=[
            pltpu.VMEM((m, hid), jnp.bfloat16),
            pltpu.VMEM((m, hid), jnp.bfloat16),
            pltpu.VMEM((hid, d_model), jnp.bfloat16),
            pltpu.VMEM((2, m, wp), jnp.float32),
            pltpu.VMEM((2, dp, d_model), jnp.float32),
            pltpu.VMEM((2, chunk, d_model), jnp.float32),
            pltpu.SemaphoreType.DMA((2,)),
            pltpu.SemaphoreType.DMA((2,)),
            pltpu.SemaphoreType.DMA((2,)),
            pltpu.VMEM((2 * n_rs, chunk, half), jnp.bfloat16),
            pltpu.SemaphoreType.DMA((n_rs,)),
            pltpu.SemaphoreType.DMA((n_rs,)),
            pltpu.SemaphoreType.DMA((n_rs,)),
            pltpu.SemaphoreType.DMA((n_rs,)),
            pltpu.SemaphoreType.DMA((n_ag,)),
            pltpu.SemaphoreType.DMA((n_ag,)),
            pltpu.SemaphoreType.DMA((n_ag,)),
            pltpu.SemaphoreType.DMA((n_ag,)),
        ],
        compiler_params=pltpu.CompilerParams(
            collective_id=0,
            vmem_limit_bytes=63 * 1024 * 1024,
        ),
    )(x, Wg, Wu, Wd)
